# baseline (speedup 1.0000x reference)
import functools

import jax
import jax.numpy as jnp
from jax import lax
from jax.experimental import pallas as pl
from jax.experimental.pallas import tpu as pltpu
from jax.experimental.pallas import tpu_sc as plsc

N = 10000
NP = 10240
F = 128
NC, NS = 2, 16
NW = NC * NS
EP_PAD = 327680
ED_PAD = 163840
SUB = 64
RING = 4
OPS_P = EP_PAD // NW // SUB
OPS_D = ED_PAD // NW // SUB
IP = 16
LAG = 3

def _zero_fill(vmem_ref, nrows):
    z = jnp.zeros((16,), jnp.float32)

    def body(k, _):
        vmem_ref[k // 8, pl.ds((k % 8) * 16, 16)] = z
        return 0

    lax.fori_loop(0, nrows * 8, body, 0)


@functools.cache
def _sc_mesh():
    return plsc.VectorSubcoreMesh(core_axis_name="c", subcore_axis_name="s",
                                  num_cores=NC, num_subcores=NS)


@functools.cache
def _deg_kernel_fn():
    return pl.kernel(
        _deg_body,
        out_type=[jax.ShapeDtypeStruct((NC, NP), jnp.float32),
                  jax.ShapeDtypeStruct((NC, NP), jnp.float32)],
        mesh=_sc_mesh(),
        scratch_types=[
            pltpu.VMEM_SHARED((NP,), jnp.float32),
            pltpu.VMEM_SHARED((NP,), jnp.float32),
            pltpu.VMEM((SUB,), jnp.float32),
            pltpu.VMEM((640,), jnp.float32),
            pltpu.VMEM((OPS_P, SUB), jnp.int32),
            pltpu.SemaphoreType.DMA,
        ],
    )


def _deg_body(dstp, dstd, degp, degd, shp, shd, ones, zb, didx, sem):
    c = lax.axis_index("c")
    s = lax.axis_index("s")
    w = c * NS + s
    o16 = jnp.ones((16,), jnp.float32)
    z16 = jnp.zeros((16,), jnp.float32)

    def init(k, _):
        ones[pl.ds(k * 16, 16)] = o16
        return 0

    lax.fori_loop(0, SUB // 16, init, 0)

    def zb2(k, _):
        zb[pl.ds(k * 16, 16)] = z16
        return 0

    lax.fori_loop(0, 40, zb2, 0)
    r0 = s * 640
    pltpu.sync_copy(zb, shp.at[pl.ds(r0, 640)])
    pltpu.sync_copy(zb, shd.at[pl.ds(r0, 640)])
    plsc.subcore_barrier()

    def count_pass(dst2, sh, nops):
        pltpu.sync_copy(dst2.at[pl.ds(w * nops, nops)],
                        didx.at[pl.ds(0, nops)])

        def chunk(cix, _):
            b0 = cix * 8
            cps = [pltpu.async_copy(ones, sh.at[didx.at[b0 + k]], sem,
                                    add=True)
                   for k in range(8)]
            for cp in cps:
                cp.wait()
            return 0

        lax.fori_loop(0, nops // 8, chunk, 0)

    count_pass(dstp, shp, OPS_P)
    count_pass(dstd, shd, OPS_D)
    plsc.subcore_barrier()
    pltpu.sync_copy(shp.at[pl.ds(r0, 640)], degp.at[c, pl.ds(r0, 640)])
    pltpu.sync_copy(shd.at[pl.ds(r0, 640)], degd.at[c, pl.ds(r0, 640)])


@functools.cache
def _conv_kernel_fn():
    return pl.kernel(
        _conv_body,
        out_type=[jax.ShapeDtypeStruct((NC, NP, F), jnp.float32),
                  jax.ShapeDtypeStruct((NC, NP, F), jnp.float32)],
        mesh=_sc_mesh(),
        scratch_types=[
            pltpu.VMEM_SHARED((NP, F), jnp.float32),
            pltpu.VMEM((3 * IP, SUB), jnp.int32),
            pltpu.VMEM((3 * IP, SUB), jnp.int32),
            pltpu.VMEM((RING, SUB, F), jnp.float32),
            pltpu.VMEM((16, F), jnp.float32),
            pltpu.SemaphoreType.DMA,
            pltpu.SemaphoreType.DMA,
            pltpu.SemaphoreType.DMA,
        ],
    )


def _conv_body(hp, srcp, dstp, hd, srcd, dstd, outp, outd,
               shacc, sidx, didx, rows, zbuf, gsem, ssem, isem):
    c = lax.axis_index("c")
    s = lax.axis_index("s")
    w = c * NS + s
    r0 = s * 640
    _zero_fill(zbuf, 16)

    def zero_acc():
        cps = [pltpu.async_copy(zbuf, shacc.at[pl.ds(r0 + k * 16, 16)], ssem)
               for k in range(40)]
        for cp in cps:
            cp.wait()

    def edge_pass(h_ref, src2, dst2, nops):
        nparts = nops // IP
        base = w * nops

        def idx_load(p):
            hb = base + p * IP
            ro = (p % 3) * IP
            return (pltpu.async_copy(src2.at[pl.ds(hb, IP)],
                                     sidx.at[pl.ds(ro, IP)], isem),
                    pltpu.async_copy(dst2.at[pl.ds(hb, IP)],
                                     didx.at[pl.ds(ro, IP)], isem))

        def fire_g(j):
            return pltpu.async_copy(h_ref.at[sidx.at[j % (3 * IP)]],
                                    rows.at[j % RING], gsem)

        def wait_g(j):
            pltpu.make_async_copy(h_ref.at[sidx.at[j % (3 * IP)]],
                                  rows.at[j % RING], gsem).wait()

        def fire_s(j):
            return pltpu.async_copy(rows.at[j % RING],
                                    shacc.at[didx.at[j % (3 * IP)]],
                                    ssem, add=True)

        def drain_s(j):
            pltpu.make_async_copy(rows.at[j % RING],
                                  shacc.at[didx.at[j % (3 * IP)]], ssem).wait()

        def step(j, _):
            drain_s(j - RING)
            fire_g(j)
            wait_g(j - LAG)
            fire_s(j - LAG)
            return 0

        cp = idx_load(0)
        cp[0].wait()
        cp[1].wait()
        nxt = idx_load(1)
        for j in range(LAG):
            fire_g(j)
        for j in range(LAG, RING):
            fire_g(j)
            wait_g(j - LAG)
            fire_s(j - LAG)
        lax.fori_loop(RING, IP, step, 0)
        for p in range(1, nparts):
            nxt[0].wait()
            nxt[1].wait()
            if p + 1 < nparts:
                nxt = idx_load(p + 1)
            lax.fori_loop(p * IP, (p + 1) * IP, step, 0)
        for j in range(nops, nops + LAG):
            drain_s(j - RING)
            wait_g(j - LAG)
            fire_s(j - LAG)
        for j in range(nops + LAG, nops + RING):
            drain_s(j - RING)

    zero_acc()
    plsc.subcore_barrier()
    edge_pass(hp, srcp, dstp, OPS_P)
    plsc.subcore_barrier()
    pltpu.sync_copy(shacc.at[pl.ds(r0, 640)], outp.at[c, pl.ds(r0, 640)])
    zero_acc()
    plsc.subcore_barrier()
    edge_pass(hd, srcd, dstd, OPS_D)
    plsc.subcore_barrier()
    pltpu.sync_copy(shacc.at[pl.ds(r0, 640)], outd.at[c, pl.ds(r0, 640)])


def _pre_body(degp_ref, px_ref, wp_ref, degd_ref, dx_ref, wd_ref,
              hp_ref, disp_ref, hd_ref, disd_ref):
    disp = lax.rsqrt(degp_ref[0] + degp_ref[1] + 1.0)
    disp_ref[...] = disp
    hp_ref[...] = jnp.dot(px_ref[...], wp_ref[...],
                          preferred_element_type=jnp.float32) * disp
    disd = lax.rsqrt(degd_ref[0] + degd_ref[1] + 1.0)
    disd_ref[...] = disd
    hd_ref[...] = jnp.dot(dx_ref[...], wd_ref[...],
                          preferred_element_type=jnp.float32) * disd


def _pre_call(deg3p, px, wp, deg3d, dx, wd):
    R = 1000
    return pl.pallas_call(
        _pre_body,
        grid=(N // R,),
        in_specs=[
            pl.BlockSpec((NC, R, 1), lambda i: (0, i, 0)),
            pl.BlockSpec((R, F), lambda i: (i, 0)),
            pl.BlockSpec((F, F), lambda i: (0, 0)),
            pl.BlockSpec((NC, R, 1), lambda i: (0, i, 0)),
            pl.BlockSpec((R, F), lambda i: (i, 0)),
            pl.BlockSpec((F, F), lambda i: (0, 0)),
        ],
        out_specs=[
            pl.BlockSpec((R, F), lambda i: (i, 0)),
            pl.BlockSpec((R, 1), lambda i: (i, 0)),
            pl.BlockSpec((R, F), lambda i: (i, 0)),
            pl.BlockSpec((R, 1), lambda i: (i, 0)),
        ],
        out_shape=[
            jax.ShapeDtypeStruct((N, F), jnp.float32),
            jax.ShapeDtypeStruct((N, 1), jnp.float32),
            jax.ShapeDtypeStruct((N, F), jnp.float32),
            jax.ShapeDtypeStruct((N, 1), jnp.float32),
        ],
    )(deg3p, px, wp, deg3d, dx, wd)


def _mid_body(ap_ref, hp_ref, disp_ref, bp_ref, wp2_ref,
              ad_ref, hd_ref, disd_ref, bd_ref, wd2_ref,
              hp2_ref, hd2_ref):
    pg = jnp.maximum(
        (ap_ref[0] + ap_ref[1] + hp_ref[...]) * disp_ref[...] + bp_ref[...], 0.0)
    hp2_ref[...] = jnp.dot(pg, wp2_ref[...],
                           preferred_element_type=jnp.float32) * disp_ref[...]
    dg = jnp.maximum(
        (ad_ref[0] + ad_ref[1] + hd_ref[...]) * disd_ref[...] + bd_ref[...], 0.0)
    hd2_ref[...] = jnp.dot(dg, wd2_ref[...],
                           preferred_element_type=jnp.float32) * disd_ref[...]


def _mid_call(ap, hp, disp, bp, wp2, ad, hd, disd, bd, wd2):
    R = 1000
    return pl.pallas_call(
        _mid_body,
        grid=(N // R,),
        in_specs=[
            pl.BlockSpec((NC, R, F), lambda i: (0, i, 0)),
            pl.BlockSpec((R, F), lambda i: (i, 0)),
            pl.BlockSpec((R, 1), lambda i: (i, 0)),
            pl.BlockSpec((1, F), lambda i: (0, 0)),
            pl.BlockSpec((F, F), lambda i: (0, 0)),
            pl.BlockSpec((NC, R, F), lambda i: (0, i, 0)),
            pl.BlockSpec((R, F), lambda i: (i, 0)),
            pl.BlockSpec((R, 1), lambda i: (i, 0)),
            pl.BlockSpec((1, F), lambda i: (0, 0)),
            pl.BlockSpec((F, F), lambda i: (0, 0)),
        ],
        out_specs=[
            pl.BlockSpec((R, F), lambda i: (i, 0)),
            pl.BlockSpec((R, F), lambda i: (i, 0)),
        ],
        out_shape=[
            jax.ShapeDtypeStruct((N, F), jnp.float32),
            jax.ShapeDtypeStruct((N, F), jnp.float32),
        ],
    )(ap, hp, disp, bp, wp2, ad, hd, disd, bd, wd2)


def _final_body(ap_ref, hp_ref, disp_ref, bp_ref,
                ad_ref, hd_ref, disd_ref, bd_ref,
                pseq_ref, wps1_ref, bps1_ref, wps2_ref, bps2_ref,
                dseq_ref, wds1_ref, bds1_ref, wds2_ref, bds2_ref,
                wfc1_ref, bfc1_ref, wfc2_ref, bfc2_ref,
                out_ref, acc_ref):
    i = pl.program_id(0)
    pg = jnp.maximum(
        (ap_ref[0] + ap_ref[1] + hp_ref[...]) * disp_ref[...] + bp_ref[...], 0.0)
    dg = jnp.maximum(
        (ad_ref[0] + ad_ref[1] + hd_ref[...]) * disd_ref[...] + bd_ref[...], 0.0)
    psum = jnp.sum(pg, axis=0, keepdims=True)
    dsum = jnp.sum(dg, axis=0, keepdims=True)

    @pl.when(i == 0)
    def _():
        acc_ref[0:1] = psum
        acc_ref[1:2] = dsum

    @pl.when(i > 0)
    def _():
        acc_ref[0:1] += psum
        acc_ref[1:2] += dsum

    @pl.when(i == pl.num_programs(0) - 1)
    def _():
        inv_n = 1.0 / N
        pgm = acc_ref[0:1] * inv_n
        dgm = acc_ref[1:2] * inv_n

        def mlp2(x, w1, b1, w2, b2):
            h = jnp.maximum(
                jnp.dot(x, w1, preferred_element_type=jnp.float32) + b1, 0.0)
            return jnp.maximum(
                jnp.dot(h, w2, preferred_element_type=jnp.float32) + b2, 0.0)

        ps = mlp2(pseq_ref[...], wps1_ref[...], bps1_ref[...],
                  wps2_ref[...], bps2_ref[...])
        ds = mlp2(dseq_ref[...], wds1_ref[...], bds1_ref[...],
                  wds2_ref[...], bds2_ref[...])
        h = jnp.maximum(
            jnp.dot(pgm, wfc1_ref[0:F], preferred_element_type=jnp.float32)
            + jnp.dot(dgm, wfc1_ref[F:2 * F], preferred_element_type=jnp.float32)
            + jnp.dot(ps, wfc1_ref[2 * F:3 * F], preferred_element_type=jnp.float32)
            + jnp.dot(ds, wfc1_ref[3 * F:4 * F], preferred_element_type=jnp.float32)
            + bfc1_ref[...], 0.0)
        out_ref[...] = (jnp.dot(h, wfc2_ref[...],
                                preferred_element_type=jnp.float32)
                        + bfc2_ref[...])


def _final_call(ap, hp, disp, bp, ad, hd, disd, bd,
                pseq, wps1, bps1, wps2, bps2, dseq, wds1, bds1, wds2, bds2,
                wfc1, bfc1, wfc2, bfc2):
    R = 1000
    full = lambda shape: pl.BlockSpec(shape, lambda i: tuple(0 for _ in shape))
    return pl.pallas_call(
        _final_body,
        grid=(N // R,),
        in_specs=[
            pl.BlockSpec((NC, R, F), lambda i: (0, i, 0)),
            pl.BlockSpec((R, F), lambda i: (i, 0)),
            pl.BlockSpec((R, 1), lambda i: (i, 0)),
            full((1, F)),
            pl.BlockSpec((NC, R, F), lambda i: (0, i, 0)),
            pl.BlockSpec((R, F), lambda i: (i, 0)),
            pl.BlockSpec((R, 1), lambda i: (i, 0)),
            full((1, F)),
            full((1, 1024)), full((1024, F)), full((1, F)), full((F, F)), full((1, F)),
            full((1, 512)), full((512, F)), full((1, F)), full((F, F)), full((1, F)),
            full((4 * F, F)), full((1, F)), full((F, 1)), full((1, 1)),
        ],
        out_specs=pl.BlockSpec((1, 1), lambda i: (0, 0)),
        out_shape=jax.ShapeDtypeStruct((1, 1), jnp.float32),
        scratch_shapes=[pltpu.VMEM((8, F), jnp.float32)],
    )(ap, hp, disp, bp, ad, hd, disd, bd,
      pseq, wps1, bps1, wps2, bps2, dseq, wds1, bds1, wds2, bds2,
      wfc1, bfc1, wfc2, bfc2)


def _pad_edges(edge_index, total):
    src = edge_index[0].astype(jnp.int32)
    dst = edge_index[1].astype(jnp.int32)
    npad = total - src.shape[0]
    pad_ids = jnp.arange(npad, dtype=jnp.int32)
    src = jnp.concatenate([src, (pad_ids * 37) % N])
    dst = jnp.concatenate([dst, N + pad_ids % (NP - N)])
    return src, dst


def kernel(protein_x, protein_edge_index, drug_x, drug_edge_index,
           protein_seq, drug_seq,
           Wp1, bp1, Wp2, bp2, Wd1, bd1, Wd2, bd2,
           Wps1, bps1, Wps2, bps2, Wds1, bds1, Wds2, bds2,
           Wfc1, bfc1, Wfc2, bfc2):
    srcp, dstp = _pad_edges(protein_edge_index, EP_PAD)
    srcd, dstd = _pad_edges(drug_edge_index, ED_PAD)
    srcp = srcp.reshape(-1, SUB)
    dstp = dstp.reshape(-1, SUB)
    srcd = srcd.reshape(-1, SUB)
    dstd = dstd.reshape(-1, SUB)

    degp, degd = _deg_kernel_fn()(dstp, dstd)
    deg3p = degp.reshape(NC, NP, 1)
    deg3d = degd.reshape(NC, NP, 1)

    hp1, disp, hd1, disd = _pre_call(deg3p, protein_x, Wp1, deg3d, drug_x, Wd1)
    ap1, ad1 = _conv_kernel_fn()(hp1, srcp, dstp, hd1, srcd, dstd)
    hp2, hd2 = _mid_call(ap1, hp1, disp, bp1.reshape(1, F), Wp2,
                         ad1, hd1, disd, bd1.reshape(1, F), Wd2)
    ap2, ad2 = _conv_kernel_fn()(hp2, srcp, dstp, hd2, srcd, dstd)
    out = _final_call(
        ap2, hp2, disp, bp2.reshape(1, F),
        ad2, hd2, disd, bd2.reshape(1, F),
        protein_seq.reshape(1, 1024), Wps1, bps1.reshape(1, F),
        Wps2, bps2.reshape(1, F),
        drug_seq.reshape(1, 512), Wds1, bds1.reshape(1, F),
        Wds2, bds2.reshape(1, F),
        Wfc1, bfc1.reshape(1, F), Wfc2, bfc2.reshape(1, 1))
    return out.reshape(1)


# speedup vs baseline: 1.0021x; 1.0021x over previous
import functools

import jax
import jax.numpy as jnp
from jax import lax
from jax.experimental import pallas as pl
from jax.experimental.pallas import tpu as pltpu
from jax.experimental.pallas import tpu_sc as plsc

N = 10000
NP = 10240
F = 128
NC, NS = 2, 16
NW = NC * NS
EP_PAD = 327680
ED_PAD = 163840
SUB = 64
RING = 4
OPS_P = EP_PAD // NW // SUB
OPS_D = ED_PAD // NW // SUB
IP = 16
LAG = 2

def _zero_fill(vmem_ref, nrows):
    z = jnp.zeros((16,), jnp.float32)

    def body(k, _):
        vmem_ref[k // 8, pl.ds((k % 8) * 16, 16)] = z
        return 0

    lax.fori_loop(0, nrows * 8, body, 0)


@functools.cache
def _sc_mesh():
    return plsc.VectorSubcoreMesh(core_axis_name="c", subcore_axis_name="s",
                                  num_cores=NC, num_subcores=NS)


@functools.cache
def _deg_kernel_fn():
    return pl.kernel(
        _deg_body,
        out_type=[jax.ShapeDtypeStruct((NC, NP), jnp.float32),
                  jax.ShapeDtypeStruct((NC, NP), jnp.float32)],
        mesh=_sc_mesh(),
        scratch_types=[
            pltpu.VMEM_SHARED((NP,), jnp.float32),
            pltpu.VMEM_SHARED((NP,), jnp.float32),
            pltpu.VMEM((SUB,), jnp.float32),
            pltpu.VMEM((640,), jnp.float32),
            pltpu.VMEM((OPS_P, SUB), jnp.int32),
            pltpu.SemaphoreType.DMA,
        ],
    )


def _deg_body(dstp, dstd, degp, degd, shp, shd, ones, zb, didx, sem):
    c = lax.axis_index("c")
    s = lax.axis_index("s")
    w = c * NS + s
    o16 = jnp.ones((16,), jnp.float32)
    z16 = jnp.zeros((16,), jnp.float32)

    def init(k, _):
        ones[pl.ds(k * 16, 16)] = o16
        return 0

    lax.fori_loop(0, SUB // 16, init, 0)

    def zb2(k, _):
        zb[pl.ds(k * 16, 16)] = z16
        return 0

    lax.fori_loop(0, 40, zb2, 0)
    r0 = s * 640
    pltpu.sync_copy(zb, shp.at[pl.ds(r0, 640)])
    pltpu.sync_copy(zb, shd.at[pl.ds(r0, 640)])
    plsc.subcore_barrier()

    def count_pass(dst2, sh, nops):
        pltpu.sync_copy(dst2.at[pl.ds(w * nops, nops)],
                        didx.at[pl.ds(0, nops)])

        def chunk(cix, _):
            b0 = cix * 8
            cps = [pltpu.async_copy(ones, sh.at[didx.at[b0 + k]], sem,
                                    add=True)
                   for k in range(8)]
            for cp in cps:
                cp.wait()
            return 0

        lax.fori_loop(0, nops // 8, chunk, 0)

    count_pass(dstp, shp, OPS_P)
    count_pass(dstd, shd, OPS_D)
    plsc.subcore_barrier()
    pltpu.sync_copy(shp.at[pl.ds(r0, 640)], degp.at[c, pl.ds(r0, 640)])
    pltpu.sync_copy(shd.at[pl.ds(r0, 640)], degd.at[c, pl.ds(r0, 640)])


@functools.cache
def _conv_kernel_fn():
    return pl.kernel(
        _conv_body,
        out_type=[jax.ShapeDtypeStruct((NC, NP, F), jnp.float32),
                  jax.ShapeDtypeStruct((NC, NP, F), jnp.float32)],
        mesh=_sc_mesh(),
        scratch_types=[
            pltpu.VMEM_SHARED((NP, F), jnp.float32),
            pltpu.VMEM((3 * IP, SUB), jnp.int32),
            pltpu.VMEM((3 * IP, SUB), jnp.int32),
            pltpu.VMEM((RING, SUB, F), jnp.float32),
            pltpu.VMEM((16, F), jnp.float32),
            pltpu.SemaphoreType.DMA,
            pltpu.SemaphoreType.DMA,
            pltpu.SemaphoreType.DMA,
        ],
    )


def _conv_body(hp, srcp, dstp, hd, srcd, dstd, outp, outd,
               shacc, sidx, didx, rows, zbuf, gsem, ssem, isem):
    c = lax.axis_index("c")
    s = lax.axis_index("s")
    w = c * NS + s
    r0 = s * 640
    _zero_fill(zbuf, 16)

    def zero_acc():
        cps = [pltpu.async_copy(zbuf, shacc.at[pl.ds(r0 + k * 16, 16)], ssem)
               for k in range(40)]
        for cp in cps:
            cp.wait()

    def edge_pass(h_ref, src2, dst2, nops):
        nparts = nops // IP
        base = w * nops

        def idx_load(p):
            hb = base + p * IP
            ro = (p % 3) * IP
            return (pltpu.async_copy(src2.at[pl.ds(hb, IP)],
                                     sidx.at[pl.ds(ro, IP)], isem),
                    pltpu.async_copy(dst2.at[pl.ds(hb, IP)],
                                     didx.at[pl.ds(ro, IP)], isem))

        def fire_g(j):
            return pltpu.async_copy(h_ref.at[sidx.at[j % (3 * IP)]],
                                    rows.at[j % RING], gsem)

        def wait_g(j):
            pltpu.make_async_copy(h_ref.at[sidx.at[j % (3 * IP)]],
                                  rows.at[j % RING], gsem).wait()

        def fire_s(j):
            return pltpu.async_copy(rows.at[j % RING],
                                    shacc.at[didx.at[j % (3 * IP)]],
                                    ssem, add=True)

        def drain_s(j):
            pltpu.make_async_copy(rows.at[j % RING],
                                  shacc.at[didx.at[j % (3 * IP)]], ssem).wait()

        def step(j, _):
            drain_s(j - RING)
            fire_g(j)
            wait_g(j - LAG)
            fire_s(j - LAG)
            return 0

        cp = idx_load(0)
        cp[0].wait()
        cp[1].wait()
        nxt = idx_load(1)
        for j in range(LAG):
            fire_g(j)
        for j in range(LAG, RING):
            fire_g(j)
            wait_g(j - LAG)
            fire_s(j - LAG)
        lax.fori_loop(RING, IP, step, 0)
        for p in range(1, nparts):
            nxt[0].wait()
            nxt[1].wait()
            if p + 1 < nparts:
                nxt = idx_load(p + 1)
            lax.fori_loop(p * IP, (p + 1) * IP, step, 0)
        for j in range(nops, nops + LAG):
            drain_s(j - RING)
            wait_g(j - LAG)
            fire_s(j - LAG)
        for j in range(nops + LAG, nops + RING):
            drain_s(j - RING)

    zero_acc()
    plsc.subcore_barrier()
    edge_pass(hp, srcp, dstp, OPS_P)
    plsc.subcore_barrier()
    pltpu.sync_copy(shacc.at[pl.ds(r0, 640)], outp.at[c, pl.ds(r0, 640)])
    zero_acc()
    plsc.subcore_barrier()
    edge_pass(hd, srcd, dstd, OPS_D)
    plsc.subcore_barrier()
    pltpu.sync_copy(shacc.at[pl.ds(r0, 640)], outd.at[c, pl.ds(r0, 640)])


def _pre_body(degp_ref, px_ref, wp_ref, degd_ref, dx_ref, wd_ref,
              hp_ref, disp_ref, hd_ref, disd_ref):
    disp = lax.rsqrt(degp_ref[0] + degp_ref[1] + 1.0)
    disp_ref[...] = disp
    hp_ref[...] = jnp.dot(px_ref[...], wp_ref[...],
                          preferred_element_type=jnp.float32) * disp
    disd = lax.rsqrt(degd_ref[0] + degd_ref[1] + 1.0)
    disd_ref[...] = disd
    hd_ref[...] = jnp.dot(dx_ref[...], wd_ref[...],
                          preferred_element_type=jnp.float32) * disd


def _pre_call(deg3p, px, wp, deg3d, dx, wd):
    R = 1000
    return pl.pallas_call(
        _pre_body,
        grid=(N // R,),
        in_specs=[
            pl.BlockSpec((NC, R, 1), lambda i: (0, i, 0)),
            pl.BlockSpec((R, F), lambda i: (i, 0)),
            pl.BlockSpec((F, F), lambda i: (0, 0)),
            pl.BlockSpec((NC, R, 1), lambda i: (0, i, 0)),
            pl.BlockSpec((R, F), lambda i: (i, 0)),
            pl.BlockSpec((F, F), lambda i: (0, 0)),
        ],
        out_specs=[
            pl.BlockSpec((R, F), lambda i: (i, 0)),
            pl.BlockSpec((R, 1), lambda i: (i, 0)),
            pl.BlockSpec((R, F), lambda i: (i, 0)),
            pl.BlockSpec((R, 1), lambda i: (i, 0)),
        ],
        out_shape=[
            jax.ShapeDtypeStruct((N, F), jnp.float32),
            jax.ShapeDtypeStruct((N, 1), jnp.float32),
            jax.ShapeDtypeStruct((N, F), jnp.float32),
            jax.ShapeDtypeStruct((N, 1), jnp.float32),
        ],
    )(deg3p, px, wp, deg3d, dx, wd)


def _mid_body(ap_ref, hp_ref, disp_ref, bp_ref, wp2_ref,
              ad_ref, hd_ref, disd_ref, bd_ref, wd2_ref,
              hp2_ref, hd2_ref):
    pg = jnp.maximum(
        (ap_ref[0] + ap_ref[1] + hp_ref[...]) * disp_ref[...] + bp_ref[...], 0.0)
    hp2_ref[...] = jnp.dot(pg, wp2_ref[...],
                           preferred_element_type=jnp.float32) * disp_ref[...]
    dg = jnp.maximum(
        (ad_ref[0] + ad_ref[1] + hd_ref[...]) * disd_ref[...] + bd_ref[...], 0.0)
    hd2_ref[...] = jnp.dot(dg, wd2_ref[...],
                           preferred_element_type=jnp.float32) * disd_ref[...]


def _mid_call(ap, hp, disp, bp, wp2, ad, hd, disd, bd, wd2):
    R = 1000
    return pl.pallas_call(
        _mid_body,
        grid=(N // R,),
        in_specs=[
            pl.BlockSpec((NC, R, F), lambda i: (0, i, 0)),
            pl.BlockSpec((R, F), lambda i: (i, 0)),
            pl.BlockSpec((R, 1), lambda i: (i, 0)),
            pl.BlockSpec((1, F), lambda i: (0, 0)),
            pl.BlockSpec((F, F), lambda i: (0, 0)),
            pl.BlockSpec((NC, R, F), lambda i: (0, i, 0)),
            pl.BlockSpec((R, F), lambda i: (i, 0)),
            pl.BlockSpec((R, 1), lambda i: (i, 0)),
            pl.BlockSpec((1, F), lambda i: (0, 0)),
            pl.BlockSpec((F, F), lambda i: (0, 0)),
        ],
        out_specs=[
            pl.BlockSpec((R, F), lambda i: (i, 0)),
            pl.BlockSpec((R, F), lambda i: (i, 0)),
        ],
        out_shape=[
            jax.ShapeDtypeStruct((N, F), jnp.float32),
            jax.ShapeDtypeStruct((N, F), jnp.float32),
        ],
    )(ap, hp, disp, bp, wp2, ad, hd, disd, bd, wd2)


def _final_body(ap_ref, hp_ref, disp_ref, bp_ref,
                ad_ref, hd_ref, disd_ref, bd_ref,
                pseq_ref, wps1_ref, bps1_ref, wps2_ref, bps2_ref,
                dseq_ref, wds1_ref, bds1_ref, wds2_ref, bds2_ref,
                wfc1_ref, bfc1_ref, wfc2_ref, bfc2_ref,
                out_ref, acc_ref):
    i = pl.program_id(0)
    pg = jnp.maximum(
        (ap_ref[0] + ap_ref[1] + hp_ref[...]) * disp_ref[...] + bp_ref[...], 0.0)
    dg = jnp.maximum(
        (ad_ref[0] + ad_ref[1] + hd_ref[...]) * disd_ref[...] + bd_ref[...], 0.0)
    psum = jnp.sum(pg, axis=0, keepdims=True)
    dsum = jnp.sum(dg, axis=0, keepdims=True)

    @pl.when(i == 0)
    def _():
        acc_ref[0:1] = psum
        acc_ref[1:2] = dsum

    @pl.when(i > 0)
    def _():
        acc_ref[0:1] += psum
        acc_ref[1:2] += dsum

    @pl.when(i == pl.num_programs(0) - 1)
    def _():
        inv_n = 1.0 / N
        pgm = acc_ref[0:1] * inv_n
        dgm = acc_ref[1:2] * inv_n

        def mlp2(x, w1, b1, w2, b2):
            h = jnp.maximum(
                jnp.dot(x, w1, preferred_element_type=jnp.float32) + b1, 0.0)
            return jnp.maximum(
                jnp.dot(h, w2, preferred_element_type=jnp.float32) + b2, 0.0)

        ps = mlp2(pseq_ref[...], wps1_ref[...], bps1_ref[...],
                  wps2_ref[...], bps2_ref[...])
        ds = mlp2(dseq_ref[...], wds1_ref[...], bds1_ref[...],
                  wds2_ref[...], bds2_ref[...])
        h = jnp.maximum(
            jnp.dot(pgm, wfc1_ref[0:F], preferred_element_type=jnp.float32)
            + jnp.dot(dgm, wfc1_ref[F:2 * F], preferred_element_type=jnp.float32)
            + jnp.dot(ps, wfc1_ref[2 * F:3 * F], preferred_element_type=jnp.float32)
            + jnp.dot(ds, wfc1_ref[3 * F:4 * F], preferred_element_type=jnp.float32)
            + bfc1_ref[...], 0.0)
        out_ref[...] = (jnp.dot(h, wfc2_ref[...],
                                preferred_element_type=jnp.float32)
                        + bfc2_ref[...])


def _final_call(ap, hp, disp, bp, ad, hd, disd, bd,
                pseq, wps1, bps1, wps2, bps2, dseq, wds1, bds1, wds2, bds2,
                wfc1, bfc1, wfc2, bfc2):
    R = 1000
    full = lambda shape: pl.BlockSpec(shape, lambda i: tuple(0 for _ in shape))
    return pl.pallas_call(
        _final_body,
        grid=(N // R,),
        in_specs=[
            pl.BlockSpec((NC, R, F), lambda i: (0, i, 0)),
            pl.BlockSpec((R, F), lambda i: (i, 0)),
            pl.BlockSpec((R, 1), lambda i: (i, 0)),
            full((1, F)),
            pl.BlockSpec((NC, R, F), lambda i: (0, i, 0)),
            pl.BlockSpec((R, F), lambda i: (i, 0)),
            pl.BlockSpec((R, 1), lambda i: (i, 0)),
            full((1, F)),
            full((1, 1024)), full((1024, F)), full((1, F)), full((F, F)), full((1, F)),
            full((1, 512)), full((512, F)), full((1, F)), full((F, F)), full((1, F)),
            full((4 * F, F)), full((1, F)), full((F, 1)), full((1, 1)),
        ],
        out_specs=pl.BlockSpec((1, 1), lambda i: (0, 0)),
        out_shape=jax.ShapeDtypeStruct((1, 1), jnp.float32),
        scratch_shapes=[pltpu.VMEM((8, F), jnp.float32)],
    )(ap, hp, disp, bp, ad, hd, disd, bd,
      pseq, wps1, bps1, wps2, bps2, dseq, wds1, bds1, wds2, bds2,
      wfc1, bfc1, wfc2, bfc2)


def _pad_edges(edge_index, total):
    src = edge_index[0].astype(jnp.int32)
    dst = edge_index[1].astype(jnp.int32)
    npad = total - src.shape[0]
    pad_ids = jnp.arange(npad, dtype=jnp.int32)
    src = jnp.concatenate([src, (pad_ids * 37) % N])
    dst = jnp.concatenate([dst, N + pad_ids % (NP - N)])
    return src, dst


def kernel(protein_x, protein_edge_index, drug_x, drug_edge_index,
           protein_seq, drug_seq,
           Wp1, bp1, Wp2, bp2, Wd1, bd1, Wd2, bd2,
           Wps1, bps1, Wps2, bps2, Wds1, bds1, Wds2, bds2,
           Wfc1, bfc1, Wfc2, bfc2):
    srcp, dstp = _pad_edges(protein_edge_index, EP_PAD)
    srcd, dstd = _pad_edges(drug_edge_index, ED_PAD)
    srcp = srcp.reshape(-1, SUB)
    dstp = dstp.reshape(-1, SUB)
    srcd = srcd.reshape(-1, SUB)
    dstd = dstd.reshape(-1, SUB)

    degp, degd = _deg_kernel_fn()(dstp, dstd)
    deg3p = degp.reshape(NC, NP, 1)
    deg3d = degd.reshape(NC, NP, 1)

    hp1, disp, hd1, disd = _pre_call(deg3p, protein_x, Wp1, deg3d, drug_x, Wd1)
    ap1, ad1 = _conv_kernel_fn()(hp1, srcp, dstp, hd1, srcd, dstd)
    hp2, hd2 = _mid_call(ap1, hp1, disp, bp1.reshape(1, F), Wp2,
                         ad1, hd1, disd, bd1.reshape(1, F), Wd2)
    ap2, ad2 = _conv_kernel_fn()(hp2, srcp, dstp, hd2, srcd, dstd)
    out = _final_call(
        ap2, hp2, disp, bp2.reshape(1, F),
        ad2, hd2, disd, bd2.reshape(1, F),
        protein_seq.reshape(1, 1024), Wps1, bps1.reshape(1, F),
        Wps2, bps2.reshape(1, F),
        drug_seq.reshape(1, 512), Wds1, bds1.reshape(1, F),
        Wds2, bds2.reshape(1, F),
        Wfc1, bfc1.reshape(1, F), Wfc2, bfc2.reshape(1, 1))
    return out.reshape(1)


# speedup vs baseline: 1.0088x; 1.0068x over previous
import functools

import jax
import jax.numpy as jnp
from jax import lax
from jax.experimental import pallas as pl
from jax.experimental.pallas import tpu as pltpu
from jax.experimental.pallas import tpu_sc as plsc

N = 10000
NP = 10240
F = 128
NC, NS = 2, 16
NW = NC * NS
EP_PAD = 327680
ED_PAD = 163840
SUB = 64
RING = 4
OPS_P = EP_PAD // NW // SUB
OPS_D = ED_PAD // NW // SUB
IP = 16
LAG = 2

def _zero_fill(vmem_ref, nrows):
    z = jnp.zeros((16,), jnp.float32)

    def body(k, _):
        vmem_ref[k // 8, pl.ds((k % 8) * 16, 16)] = z
        return 0

    lax.fori_loop(0, nrows * 8, body, 0)


@functools.cache
def _sc_mesh():
    return plsc.VectorSubcoreMesh(core_axis_name="c", subcore_axis_name="s",
                                  num_cores=NC, num_subcores=NS)


@functools.cache
def _deg_kernel_fn():
    return pl.kernel(
        _deg_body,
        out_type=[jax.ShapeDtypeStruct((NC, NP), jnp.float32),
                  jax.ShapeDtypeStruct((NC, NP), jnp.float32)],
        mesh=_sc_mesh(),
        scratch_types=[
            pltpu.VMEM_SHARED((NP,), jnp.float32),
            pltpu.VMEM_SHARED((NP,), jnp.float32),
            pltpu.VMEM((SUB,), jnp.float32),
            pltpu.VMEM((640,), jnp.float32),
            pltpu.VMEM((OPS_P, SUB), jnp.int32),
            pltpu.SemaphoreType.DMA,
        ],
    )


def _deg_body(dstp, dstd, degp, degd, shp, shd, ones, zb, didx, sem):
    c = lax.axis_index("c")
    s = lax.axis_index("s")
    w = c * NS + s
    o16 = jnp.ones((16,), jnp.float32)
    z16 = jnp.zeros((16,), jnp.float32)

    def init(k, _):
        ones[pl.ds(k * 16, 16)] = o16
        return 0

    lax.fori_loop(0, SUB // 16, init, 0)

    def zb2(k, _):
        zb[pl.ds(k * 16, 16)] = z16
        return 0

    lax.fori_loop(0, 40, zb2, 0)
    r0 = s * 640
    pltpu.sync_copy(zb, shp.at[pl.ds(r0, 640)])
    pltpu.sync_copy(zb, shd.at[pl.ds(r0, 640)])
    plsc.subcore_barrier()

    def count_pass(dst2, sh, nops):
        D = 16
        pltpu.sync_copy(dst2.at[pl.ds(w * nops, nops)],
                        didx.at[pl.ds(0, nops)])
        for j in range(D):
            pltpu.async_copy(ones, sh.at[didx.at[j]], sem, add=True)

        def step(j, _):
            pltpu.make_async_copy(ones, sh.at[didx.at[j - D]], sem).wait()
            pltpu.async_copy(ones, sh.at[didx.at[j]], sem, add=True)
            return 0

        lax.fori_loop(D, nops, step, 0)
        for j in range(nops - D, nops):
            pltpu.make_async_copy(ones, sh.at[didx.at[j]], sem).wait()

    count_pass(dstp, shp, OPS_P)
    count_pass(dstd, shd, OPS_D)
    plsc.subcore_barrier()
    pltpu.sync_copy(shp.at[pl.ds(r0, 640)], degp.at[c, pl.ds(r0, 640)])
    pltpu.sync_copy(shd.at[pl.ds(r0, 640)], degd.at[c, pl.ds(r0, 640)])


@functools.cache
def _conv_kernel_fn():
    return pl.kernel(
        _conv_body,
        out_type=[jax.ShapeDtypeStruct((NC, NP, F), jnp.float32),
                  jax.ShapeDtypeStruct((NC, NP, F), jnp.float32)],
        mesh=_sc_mesh(),
        scratch_types=[
            pltpu.VMEM_SHARED((NP, F), jnp.float32),
            pltpu.VMEM((3 * IP, SUB), jnp.int32),
            pltpu.VMEM((3 * IP, SUB), jnp.int32),
            pltpu.VMEM((RING, SUB, F), jnp.float32),
            pltpu.VMEM((16, F), jnp.float32),
            pltpu.SemaphoreType.DMA,
            pltpu.SemaphoreType.DMA,
            pltpu.SemaphoreType.DMA,
        ],
    )


def _conv_body(hp, srcp, dstp, hd, srcd, dstd, outp, outd,
               shacc, sidx, didx, rows, zbuf, gsem, ssem, isem):
    c = lax.axis_index("c")
    s = lax.axis_index("s")
    w = c * NS + s
    r0 = s * 640
    _zero_fill(zbuf, 16)

    def zero_acc():
        cps = [pltpu.async_copy(zbuf, shacc.at[pl.ds(r0 + k * 16, 16)], ssem)
               for k in range(40)]
        for cp in cps:
            cp.wait()

    def edge_pass(h_ref, src2, dst2, nops):
        nparts = nops // IP
        base = w * nops

        def idx_load(p):
            hb = base + p * IP
            ro = (p % 3) * IP
            return (pltpu.async_copy(src2.at[pl.ds(hb, IP)],
                                     sidx.at[pl.ds(ro, IP)], isem),
                    pltpu.async_copy(dst2.at[pl.ds(hb, IP)],
                                     didx.at[pl.ds(ro, IP)], isem))

        def fire_g(j):
            return pltpu.async_copy(h_ref.at[sidx.at[j % (3 * IP)]],
                                    rows.at[j % RING], gsem)

        def wait_g(j):
            pltpu.make_async_copy(h_ref.at[sidx.at[j % (3 * IP)]],
                                  rows.at[j % RING], gsem).wait()

        def fire_s(j):
            return pltpu.async_copy(rows.at[j % RING],
                                    shacc.at[didx.at[j % (3 * IP)]],
                                    ssem, add=True)

        def drain_s(j):
            pltpu.make_async_copy(rows.at[j % RING],
                                  shacc.at[didx.at[j % (3 * IP)]], ssem).wait()

        def step(j, _):
            drain_s(j - RING)
            fire_g(j)
            wait_g(j - LAG)
            fire_s(j - LAG)
            return 0

        cp = idx_load(0)
        cp[0].wait()
        cp[1].wait()
        nxt = idx_load(1)
        for j in range(LAG):
            fire_g(j)
        for j in range(LAG, RING):
            fire_g(j)
            wait_g(j - LAG)
            fire_s(j - LAG)
        lax.fori_loop(RING, IP, step, 0)
        for p in range(1, nparts):
            nxt[0].wait()
            nxt[1].wait()
            if p + 1 < nparts:
                nxt = idx_load(p + 1)
            lax.fori_loop(p * IP, (p + 1) * IP, step, 0)
        for j in range(nops, nops + LAG):
            drain_s(j - RING)
            wait_g(j - LAG)
            fire_s(j - LAG)
        for j in range(nops + LAG, nops + RING):
            drain_s(j - RING)

    zero_acc()
    plsc.subcore_barrier()
    edge_pass(hp, srcp, dstp, OPS_P)
    plsc.subcore_barrier()
    pltpu.sync_copy(shacc.at[pl.ds(r0, 640)], outp.at[c, pl.ds(r0, 640)])
    zero_acc()
    plsc.subcore_barrier()
    edge_pass(hd, srcd, dstd, OPS_D)
    plsc.subcore_barrier()
    pltpu.sync_copy(shacc.at[pl.ds(r0, 640)], outd.at[c, pl.ds(r0, 640)])


def _pre_body(degp_ref, px_ref, wp_ref, degd_ref, dx_ref, wd_ref,
              hp_ref, disp_ref, hd_ref, disd_ref):
    disp = lax.rsqrt(degp_ref[0] + degp_ref[1] + 1.0)
    disp_ref[...] = disp
    hp_ref[...] = jnp.dot(px_ref[...], wp_ref[...],
                          preferred_element_type=jnp.float32) * disp
    disd = lax.rsqrt(degd_ref[0] + degd_ref[1] + 1.0)
    disd_ref[...] = disd
    hd_ref[...] = jnp.dot(dx_ref[...], wd_ref[...],
                          preferred_element_type=jnp.float32) * disd


def _pre_call(deg3p, px, wp, deg3d, dx, wd):
    R = 1000
    return pl.pallas_call(
        _pre_body,
        grid=(N // R,),
        in_specs=[
            pl.BlockSpec((NC, R, 1), lambda i: (0, i, 0)),
            pl.BlockSpec((R, F), lambda i: (i, 0)),
            pl.BlockSpec((F, F), lambda i: (0, 0)),
            pl.BlockSpec((NC, R, 1), lambda i: (0, i, 0)),
            pl.BlockSpec((R, F), lambda i: (i, 0)),
            pl.BlockSpec((F, F), lambda i: (0, 0)),
        ],
        out_specs=[
            pl.BlockSpec((R, F), lambda i: (i, 0)),
            pl.BlockSpec((R, 1), lambda i: (i, 0)),
            pl.BlockSpec((R, F), lambda i: (i, 0)),
            pl.BlockSpec((R, 1), lambda i: (i, 0)),
        ],
        out_shape=[
            jax.ShapeDtypeStruct((N, F), jnp.float32),
            jax.ShapeDtypeStruct((N, 1), jnp.float32),
            jax.ShapeDtypeStruct((N, F), jnp.float32),
            jax.ShapeDtypeStruct((N, 1), jnp.float32),
        ],
    )(deg3p, px, wp, deg3d, dx, wd)


def _mid_body(ap_ref, hp_ref, disp_ref, bp_ref, wp2_ref,
              ad_ref, hd_ref, disd_ref, bd_ref, wd2_ref,
              hp2_ref, hd2_ref):
    pg = jnp.maximum(
        (ap_ref[0] + ap_ref[1] + hp_ref[...]) * disp_ref[...] + bp_ref[...], 0.0)
    hp2_ref[...] = jnp.dot(pg, wp2_ref[...],
                           preferred_element_type=jnp.float32) * disp_ref[...]
    dg = jnp.maximum(
        (ad_ref[0] + ad_ref[1] + hd_ref[...]) * disd_ref[...] + bd_ref[...], 0.0)
    hd2_ref[...] = jnp.dot(dg, wd2_ref[...],
                           preferred_element_type=jnp.float32) * disd_ref[...]


def _mid_call(ap, hp, disp, bp, wp2, ad, hd, disd, bd, wd2):
    R = 1000
    return pl.pallas_call(
        _mid_body,
        grid=(N // R,),
        in_specs=[
            pl.BlockSpec((NC, R, F), lambda i: (0, i, 0)),
            pl.BlockSpec((R, F), lambda i: (i, 0)),
            pl.BlockSpec((R, 1), lambda i: (i, 0)),
            pl.BlockSpec((1, F), lambda i: (0, 0)),
            pl.BlockSpec((F, F), lambda i: (0, 0)),
            pl.BlockSpec((NC, R, F), lambda i: (0, i, 0)),
            pl.BlockSpec((R, F), lambda i: (i, 0)),
            pl.BlockSpec((R, 1), lambda i: (i, 0)),
            pl.BlockSpec((1, F), lambda i: (0, 0)),
            pl.BlockSpec((F, F), lambda i: (0, 0)),
        ],
        out_specs=[
            pl.BlockSpec((R, F), lambda i: (i, 0)),
            pl.BlockSpec((R, F), lambda i: (i, 0)),
        ],
        out_shape=[
            jax.ShapeDtypeStruct((N, F), jnp.float32),
            jax.ShapeDtypeStruct((N, F), jnp.float32),
        ],
    )(ap, hp, disp, bp, wp2, ad, hd, disd, bd, wd2)


def _final_body(ap_ref, hp_ref, disp_ref, bp_ref,
                ad_ref, hd_ref, disd_ref, bd_ref,
                pseq_ref, wps1_ref, bps1_ref, wps2_ref, bps2_ref,
                dseq_ref, wds1_ref, bds1_ref, wds2_ref, bds2_ref,
                wfc1_ref, bfc1_ref, wfc2_ref, bfc2_ref,
                out_ref, acc_ref):
    i = pl.program_id(0)
    pg = jnp.maximum(
        (ap_ref[0] + ap_ref[1] + hp_ref[...]) * disp_ref[...] + bp_ref[...], 0.0)
    dg = jnp.maximum(
        (ad_ref[0] + ad_ref[1] + hd_ref[...]) * disd_ref[...] + bd_ref[...], 0.0)
    psum = jnp.sum(pg, axis=0, keepdims=True)
    dsum = jnp.sum(dg, axis=0, keepdims=True)

    @pl.when(i == 0)
    def _():
        acc_ref[0:1] = psum
        acc_ref[1:2] = dsum

    @pl.when(i > 0)
    def _():
        acc_ref[0:1] += psum
        acc_ref[1:2] += dsum

    @pl.when(i == pl.num_programs(0) - 1)
    def _():
        inv_n = 1.0 / N
        pgm = acc_ref[0:1] * inv_n
        dgm = acc_ref[1:2] * inv_n

        def mlp2(x, w1, b1, w2, b2):
            h = jnp.maximum(
                jnp.dot(x, w1, preferred_element_type=jnp.float32) + b1, 0.0)
            return jnp.maximum(
                jnp.dot(h, w2, preferred_element_type=jnp.float32) + b2, 0.0)

        ps = mlp2(pseq_ref[...], wps1_ref[...], bps1_ref[...],
                  wps2_ref[...], bps2_ref[...])
        ds = mlp2(dseq_ref[...], wds1_ref[...], bds1_ref[...],
                  wds2_ref[...], bds2_ref[...])
        h = jnp.maximum(
            jnp.dot(pgm, wfc1_ref[0:F], preferred_element_type=jnp.float32)
            + jnp.dot(dgm, wfc1_ref[F:2 * F], preferred_element_type=jnp.float32)
            + jnp.dot(ps, wfc1_ref[2 * F:3 * F], preferred_element_type=jnp.float32)
            + jnp.dot(ds, wfc1_ref[3 * F:4 * F], preferred_element_type=jnp.float32)
            + bfc1_ref[...], 0.0)
        out_ref[...] = (jnp.dot(h, wfc2_ref[...],
                                preferred_element_type=jnp.float32)
                        + bfc2_ref[...])


def _final_call(ap, hp, disp, bp, ad, hd, disd, bd,
                pseq, wps1, bps1, wps2, bps2, dseq, wds1, bds1, wds2, bds2,
                wfc1, bfc1, wfc2, bfc2):
    R = 1000
    full = lambda shape: pl.BlockSpec(shape, lambda i: tuple(0 for _ in shape))
    return pl.pallas_call(
        _final_body,
        grid=(N // R,),
        in_specs=[
            pl.BlockSpec((NC, R, F), lambda i: (0, i, 0)),
            pl.BlockSpec((R, F), lambda i: (i, 0)),
            pl.BlockSpec((R, 1), lambda i: (i, 0)),
            full((1, F)),
            pl.BlockSpec((NC, R, F), lambda i: (0, i, 0)),
            pl.BlockSpec((R, F), lambda i: (i, 0)),
            pl.BlockSpec((R, 1), lambda i: (i, 0)),
            full((1, F)),
            full((1, 1024)), full((1024, F)), full((1, F)), full((F, F)), full((1, F)),
            full((1, 512)), full((512, F)), full((1, F)), full((F, F)), full((1, F)),
            full((4 * F, F)), full((1, F)), full((F, 1)), full((1, 1)),
        ],
        out_specs=pl.BlockSpec((1, 1), lambda i: (0, 0)),
        out_shape=jax.ShapeDtypeStruct((1, 1), jnp.float32),
        scratch_shapes=[pltpu.VMEM((8, F), jnp.float32)],
    )(ap, hp, disp, bp, ad, hd, disd, bd,
      pseq, wps1, bps1, wps2, bps2, dseq, wds1, bds1, wds2, bds2,
      wfc1, bfc1, wfc2, bfc2)


def _pad_edges(edge_index, total):
    src = edge_index[0].astype(jnp.int32)
    dst = edge_index[1].astype(jnp.int32)
    npad = total - src.shape[0]
    pad_ids = jnp.arange(npad, dtype=jnp.int32)
    src = jnp.concatenate([src, (pad_ids * 37) % N])
    dst = jnp.concatenate([dst, N + pad_ids % (NP - N)])
    return src, dst


def kernel(protein_x, protein_edge_index, drug_x, drug_edge_index,
           protein_seq, drug_seq,
           Wp1, bp1, Wp2, bp2, Wd1, bd1, Wd2, bd2,
           Wps1, bps1, Wps2, bps2, Wds1, bds1, Wds2, bds2,
           Wfc1, bfc1, Wfc2, bfc2):
    srcp, dstp = _pad_edges(protein_edge_index, EP_PAD)
    srcd, dstd = _pad_edges(drug_edge_index, ED_PAD)
    srcp = srcp.reshape(-1, SUB)
    dstp = dstp.reshape(-1, SUB)
    srcd = srcd.reshape(-1, SUB)
    dstd = dstd.reshape(-1, SUB)

    degp, degd = _deg_kernel_fn()(dstp, dstd)
    deg3p = degp.reshape(NC, NP, 1)
    deg3d = degd.reshape(NC, NP, 1)

    hp1, disp, hd1, disd = _pre_call(deg3p, protein_x, Wp1, deg3d, drug_x, Wd1)
    ap1, ad1 = _conv_kernel_fn()(hp1, srcp, dstp, hd1, srcd, dstd)
    hp2, hd2 = _mid_call(ap1, hp1, disp, bp1.reshape(1, F), Wp2,
                         ad1, hd1, disd, bd1.reshape(1, F), Wd2)
    ap2, ad2 = _conv_kernel_fn()(hp2, srcp, dstp, hd2, srcd, dstd)
    out = _final_call(
        ap2, hp2, disp, bp2.reshape(1, F),
        ad2, hd2, disd, bd2.reshape(1, F),
        protein_seq.reshape(1, 1024), Wps1, bps1.reshape(1, F),
        Wps2, bps2.reshape(1, F),
        drug_seq.reshape(1, 512), Wds1, bds1.reshape(1, F),
        Wds2, bds2.reshape(1, F),
        Wfc1, bfc1.reshape(1, F), Wfc2, bfc2.reshape(1, 1))
    return out.reshape(1)
